# Initial kernel scaffold; baseline (speedup 1.0000x reference)
import jax
import jax.numpy as jnp
from jax.experimental import pallas as pl


def kernel(edge_index, edge_weight, user_emb, faker_emb, item_emb):
    raise NotImplementedError("write your pallas kernel here")


# speedup vs baseline: 2.9963x; 2.9963x over previous
import functools
import jax
import jax.numpy as jnp
from jax import lax
from jax.experimental import pallas as pl
from jax.experimental.pallas import tpu as pltpu
from jax.experimental.pallas import tpu_sc as plsc

NUM_USERS = 30000
NUM_FAKERS = 500
NUM_ITEMS = 19500
N = NUM_USERS + NUM_FAKERS + NUM_ITEMS
E = 800000
D = 64
N_LAYERS = 3

NC, NS, LANES = 2, 16, 16
HALF = 25088
TRASH = 25000
NPAD = 2 * HALF
G = 128
NGC = 16
EDGE_ROWS = 6400
GPT = EDGE_ROWS // NS
NCH = GPT // NGC
RPT = HALF // NS
DV = D // LANES

_mesh = plsc.VectorSubcoreMesh(
    core_axis_name="c", subcore_axis_name="s", num_cores=NC, num_subcores=NS)


def _layer_body(src_hbm, dst_hbm, w_hbm, tab_in, tab_out,
                acc, src_ch, dst_ch, w_ch, rows, zbuf, sem0, sem1):
    c = lax.axis_index("c")
    s = lax.axis_index("s")
    lo = c * 25000
    hi = lo + 25000

    zeros16 = jnp.zeros((LANES,), jnp.float32)

    @pl.loop(0, 128)
    def _(r):
        for d in range(DV):
            zbuf[r, pl.ds(d * LANES, LANES)] = zeros16

    t0 = s * RPT
    for i in range(12):
        pltpu.sync_copy(zbuf, acc.at[pl.ds(t0 + i * 128, 128)])
    pltpu.sync_copy(zbuf.at[pl.ds(0, RPT - 12 * 128)],
                    acc.at[pl.ds(t0 + 12 * 128, RPT - 12 * 128)])
    plsc.subcore_barrier()

    row_base = s * GPT

    @pl.loop(0, NCH)
    def _(ch):
        r0 = row_base + ch * NGC
        pltpu.sync_copy(src_hbm.at[pl.ds(r0, NGC)], src_ch)
        pltpu.sync_copy(dst_hbm.at[pl.ds(r0, NGC)], dst_ch)
        pltpu.sync_copy(w_hbm.at[pl.ds(r0 * G, NGC * G)], w_ch)

        for j in range(NGC):
            for k in range(G // LANES):
                v = dst_ch[j, pl.ds(k * LANES, LANES)]
                ok = (v >= lo) & (v < hi)
                dst_ch[j, pl.ds(k * LANES, LANES)] = jnp.where(ok, v - lo, TRASH)

        cps = [None, None]
        sems = [sem0, sem1]
        cps[0] = pltpu.async_copy(tab_in.at[src_ch.at[0]], rows.at[0], sems[0])
        for g in range(NGC):
            b = g % 2
            cps[b].wait()
            if g + 1 < NGC:
                nb = (g + 1) % 2
                cps[nb] = pltpu.async_copy(
                    tab_in.at[src_ch.at[g + 1]], rows.at[nb], sems[nb])

            @pl.loop(0, G // LANES)
            def _(q):
                w16 = w_ch[pl.ds(g * G + q * LANES, LANES)]
                for l in range(LANES):
                    ws = w16[l]
                    e = q * LANES + l
                    for d in range(DV):
                        sl = pl.ds(d * LANES, LANES)
                        rows[b, e, sl] = rows[b, e, sl] * ws

            pltpu.sync_copy(rows.at[b], acc.at[dst_ch.at[g]], add=True)

    plsc.subcore_barrier()
    pltpu.sync_copy(acc.at[pl.ds(t0, RPT)],
                    tab_out.at[pl.ds(c * HALF + t0, RPT)])


_layer = pl.kernel(
    _layer_body,
    out_type=jax.ShapeDtypeStruct((NPAD, D), jnp.float32),
    mesh=_mesh,
    compiler_params=pltpu.CompilerParams(use_tc_tiling_on_sc=False),
    scratch_types=[
        pltpu.VMEM_SHARED((HALF, D), jnp.float32),
        pltpu.VMEM((NGC, G), jnp.int32),
        pltpu.VMEM((NGC, G), jnp.int32),
        pltpu.VMEM((NGC * G,), jnp.float32),
        pltpu.VMEM((2, G, D), jnp.float32),
        pltpu.VMEM((128, D), jnp.float32),
        pltpu.SemaphoreType.DMA,
        pltpu.SemaphoreType.DMA,
    ],
)


def _mean_body(t0h, t1h, t2h, t3h, out, b0, b1, b2, b3):
    c = lax.axis_index("c")
    s = lax.axis_index("s")
    w = s * NC + c
    rpt = NPAD // (NC * NS)
    base = w * rpt
    quarter = jnp.float32(0.25)

    nfull = rpt // 128
    for i in range(nfull + 1):
        sz = 128 if i < nfull else rpt - nfull * 128
        if sz == 0:
            break
        off = base + i * 128
        pltpu.sync_copy(t0h.at[pl.ds(off, sz)], b0.at[pl.ds(0, sz)])
        pltpu.sync_copy(t1h.at[pl.ds(off, sz)], b1.at[pl.ds(0, sz)])
        pltpu.sync_copy(t2h.at[pl.ds(off, sz)], b2.at[pl.ds(0, sz)])
        pltpu.sync_copy(t3h.at[pl.ds(off, sz)], b3.at[pl.ds(0, sz)])

        @pl.loop(0, sz)
        def _(r):
            for d in range(DV):
                sl = pl.ds(d * LANES, LANES)
                b0[r, sl] = (b0[r, sl] + b1[r, sl] + b2[r, sl] + b3[r, sl]) * quarter

        pltpu.sync_copy(b0.at[pl.ds(0, sz)], out.at[pl.ds(off, sz)])


_mean = pl.kernel(
    _mean_body,
    out_type=jax.ShapeDtypeStruct((NPAD, D), jnp.float32),
    mesh=_mesh,
    scratch_types=[pltpu.VMEM((128, D), jnp.float32)] * 4,
)


@jax.jit
def kernel(edge_index, edge_weight, user_emb, faker_emb, item_emb):
    src = edge_index[0].astype(jnp.int32)
    dst = edge_index[1].astype(jnp.int32)
    w = edge_weight.astype(jnp.float32)

    src = src + jnp.where(src >= 25000, 88, 0).astype(jnp.int32)
    epad = EDGE_ROWS * G - E
    src = jnp.concatenate([src, jnp.zeros((epad,), jnp.int32)]).reshape(EDGE_ROWS, G)
    dst = jnp.concatenate([dst, jnp.full((epad,), N, jnp.int32)]).reshape(EDGE_ROWS, G)
    w = jnp.concatenate([w, jnp.zeros((epad,), jnp.float32)])

    emb = jnp.concatenate([user_emb, faker_emb, item_emb], axis=0)
    tab = jnp.zeros((NPAD, D), jnp.float32)
    tab = tab.at[:25000].set(emb[:25000]).at[HALF:HALF + 25000].set(emb[25000:])

    tabs = [tab]
    for _ in range(N_LAYERS):
        tabs.append(_layer(src, dst, w, tabs[-1]))

    mean = _mean(*tabs)
    light = jnp.concatenate([mean[:25000], mean[HALF:HALF + 25000]], axis=0)
    n_user = NUM_USERS + NUM_FAKERS
    return (light[:n_user], light[n_user:])
